# baseline (speedup 1.0000x reference)
import functools

import jax
import jax.numpy as jnp
from jax import lax
from jax.experimental import pallas as pl
from jax.experimental.pallas import tpu as pltpu
from jax.experimental.pallas import tpu_sc as plsc

_U = 128


@functools.lru_cache(maxsize=None)
def _make_gather(V: int, D: int, H: int, B: int):
    info = plsc.get_sparse_core_info()
    NC, NS = info.num_cores, info.num_subcores
    NW = NC * NS
    n_blk = B // _U
    n_units_tot = H * n_blk
    assert n_units_tot % NW == 0
    n_units = n_units_tot // NW
    per_w = n_units * _U
    DT = D // 8
    mesh = plsc.VectorSubcoreMesh(core_axis_name="c", subcore_axis_name="s")

    @functools.partial(
        pl.kernel,
        mesh=mesh,
        out_type=jax.ShapeDtypeStruct((H, DT, n_blk, 8 * _U), jnp.float32),
        scratch_types=[
            pltpu.VMEM((per_w,), jnp.int32),
            pltpu.VMEM((_U, D), jnp.float32),
            pltpu.VMEM((_U, D), jnp.float32),
            pltpu.VMEM((DT, 8 * _U), jnp.float32),
            pltpu.VMEM((DT, 8 * _U), jnp.float32),
            pltpu.SemaphoreType.DMA,
            pltpu.SemaphoreType.DMA,
            pltpu.SemaphoreType.DMA,
            pltpu.SemaphoreType.DMA,
        ],
        compiler_params=pltpu.CompilerParams(
            use_tc_tiling_on_sc=False, needs_layout_passes=False),
    )
    def gather_kernel(table_hbm, idx_hbm, out_hbm, idx_v, r0, r1, t0, t1,
                      gs0, gs1, ss0, ss1):
        wid = lax.axis_index("s") * NC + lax.axis_index("c")
        pltpu.sync_copy(idx_hbm.at[wid], idx_v)
        u0 = wid * n_units

        iotas = [lax.iota(jnp.int32, 16) + 16 * k for k in range(_U // 16)]
        cols = [jnp.full((16,), e, jnp.int32) for e in range(D)]

        def g_start(t, r, s):
            pltpu.async_copy(table_hbm.at[idx_v.at[pl.ds(t * _U, _U)]], r, s)

        def g_wait(t, r, s):
            pltpu.make_async_copy(
                table_hbm.at[idx_v.at[pl.ds(t * _U, _U)]], r, s).wait()

        def dst(t):
            u = u0 + t
            return out_hbm.at[u // n_blk, :, u % n_blk]

        def s_start(t, rt, s):
            pltpu.async_copy(rt, dst(t), s)

        def s_wait(t, rt, s):
            pltpu.make_async_copy(rt, dst(t), s).wait()

        def transpose(r, rt):
            for e in range(D):
                et, es = divmod(e, 8)
                for k in range(_U // 16):
                    v = plsc.load_gather(r, [iotas[k], cols[e]])
                    rt[et, pl.ds(es * _U + 16 * k, 16)] = v

        g_start(0, r0, gs0)
        g_start(1, r1, gs1)
        g_wait(0, r0, gs0)
        transpose(r0, t0)
        s_start(0, t0, ss0)
        g_start(2, r0, gs0)
        g_wait(1, r1, gs1)
        transpose(r1, t1)
        s_start(1, t1, ss1)

        def body(io, carry):
            t = 2 + 2 * io
            g_start(t + 1, r1, gs1)
            g_wait(t, r0, gs0)
            s_wait(t - 2, t0, ss0)
            transpose(r0, t0)
            s_start(t, t0, ss0)
            g_start(t + 2, r0, gs0)
            g_wait(t + 1, r1, gs1)
            s_wait(t - 1, t1, ss1)
            transpose(r1, t1)
            s_start(t + 1, t1, ss1)
            return carry

        lax.fori_loop(0, (n_units - 4) // 2, body, 0)

        t = n_units - 2
        g_start(t + 1, r1, gs1)
        g_wait(t, r0, gs0)
        s_wait(t - 2, t0, ss0)
        transpose(r0, t0)
        s_start(t, t0, ss0)
        g_wait(t + 1, r1, gs1)
        s_wait(t - 1, t1, ss1)
        transpose(r1, t1)
        s_start(t + 1, t1, ss1)
        s_wait(t, t0, ss0)
        s_wait(t + 1, t1, ss1)

    return gather_kernel


def kernel(x, table):
    B0, H = x.shape
    V, D = table.shape
    info = plsc.get_sparse_core_info()
    NW = info.num_cores * info.num_subcores
    n_units = H * (B0 // _U) // NW
    xT = jnp.transpose(x).reshape(NW, n_units * _U).astype(jnp.int32)
    out4 = _make_gather(V, D, H, B0)(table, xT)
    out = (out4.reshape(H, D // 8, B0 // _U, 8, _U)
           .transpose(2, 4, 0, 1, 3)
           .reshape(B0, H, D))
    return out


# speedup vs baseline: 1.7160x; 1.7160x over previous
import functools

import jax
import jax.numpy as jnp
from jax import lax
from jax.experimental import pallas as pl
from jax.experimental.pallas import tpu as pltpu
from jax.experimental.pallas import tpu_sc as plsc

_BLK = 8192
_QRT = 2048


def _pack_table_body(in_ref, out_ref):
    x = in_ref[...]
    parts = [x[:, q * _QRT:(q + 1) * _QRT].T for q in range(4)]
    out_ref[...] = jnp.concatenate(parts, axis=1)


@functools.lru_cache(maxsize=None)
def _make_pack_table(V: int, D: int):
    grid = (V + _BLK - 1) // _BLK
    return pl.pallas_call(
        _pack_table_body,
        grid=(grid,),
        in_specs=[pl.BlockSpec((D, _BLK), lambda i: (0, i))],
        out_specs=pl.BlockSpec((_QRT, 128), lambda i: (i, 0)),
        out_shape=jax.ShapeDtypeStruct((grid * _QRT, 128), jnp.float32),
    )


@functools.lru_cache(maxsize=None)
def _make_gather(B: int, VP: int, D: int, C: int):
    info = plsc.get_sparse_core_info()
    NC, NS = info.num_cores, info.num_subcores
    NW = NC * NS
    assert B % NW == 0
    b_per_w = B // NW
    assert b_per_w % C == 0
    n_chunks = b_per_w // C
    assert n_chunks % 2 == 0 and n_chunks >= 4
    mesh = plsc.VectorSubcoreMesh(core_axis_name="c", subcore_axis_name="s")

    @functools.partial(
        pl.kernel,
        mesh=mesh,
        out_type=jax.ShapeDtypeStruct((B, D), jnp.float32),
        scratch_types=[
            pltpu.VMEM((n_chunks, C), jnp.int32),
            pltpu.VMEM((C, D), jnp.float32),
            pltpu.VMEM((C, D), jnp.float32),
            pltpu.SemaphoreType.DMA,
            pltpu.SemaphoreType.DMA,
            pltpu.SemaphoreType.DMA,
            pltpu.SemaphoreType.DMA,
        ],
        compiler_params=pltpu.CompilerParams(
            use_tc_tiling_on_sc=False, needs_layout_passes=False),
    )
    def gather_kernel(table_hbm, idx_hbm, out_hbm, idx_v, r0, r1, gs0, gs1,
                      ss0, ss1):
        wid = lax.axis_index("s") * NC + lax.axis_index("c")
        base = wid * b_per_w
        pltpu.sync_copy(idx_hbm.at[wid], idx_v)

        def g_start(g, r, s):
            pltpu.async_copy(table_hbm.at[idx_v.at[g]], r, s)

        def g_wait(g, r, s):
            pltpu.make_async_copy(table_hbm.at[idx_v.at[g]], r, s).wait()

        def s_start(g, r, s):
            pltpu.async_copy(r, out_hbm.at[pl.ds(base + g * C, C)], s)

        def s_wait(g, r, s):
            pltpu.make_async_copy(r, out_hbm.at[pl.ds(base + g * C, C)],
                                  s).wait()

        g_start(0, r0, gs0)
        g_start(1, r1, gs1)
        g_wait(0, r0, gs0)
        s_start(0, r0, ss0)

        def body(io, carry):
            g = 1 + 2 * io
            s_wait(g - 1, r0, ss0)
            g_start(g + 1, r0, gs0)
            g_wait(g, r1, gs1)
            s_start(g, r1, ss1)
            s_wait(g, r1, ss1)
            g_start(g + 2, r1, gs1)
            g_wait(g + 1, r0, gs0)
            s_start(g + 1, r0, ss0)
            return carry

        lax.fori_loop(0, (n_chunks - 2) // 2, body, 0)

        g = n_chunks - 1
        g_wait(g, r1, gs1)
        s_start(g, r1, ss1)
        s_wait(g - 1, r0, ss0)
        s_wait(g, r1, ss1)

    return gather_kernel


def kernel(x, table):
    B0, H = x.shape
    V, D = table.shape
    B = B0 * H
    C = 1600
    info = plsc.get_sparse_core_info()
    NW = info.num_cores * info.num_subcores
    n_chunks = B // NW // C

    table_packed = _make_pack_table(V, D)(jnp.transpose(table))
    VP = table_packed.shape[0] * (128 // D)
    table_rm = table_packed.reshape(VP, D)

    r = jnp.transpose(x).astype(jnp.int32)
    ri = (r & -8192) | ((r & 2047) << 2) | ((r >> 11) & 3)
    xi = ri.reshape(NW, n_chunks, C)

    out_lin = _make_gather(B, VP, D, C)(table_rm, xi)

    out = jnp.swapaxes(out_lin.reshape(H, B0, D), 0, 1)
    return out
